# baseline (speedup 1.0000x reference)
import jax
import jax.numpy as jnp
from jax.experimental import pallas as pl
from jax.experimental.pallas import tpu as pltpu

BK = 256
NBUF = 3


def _body(xi_h, xj_h, gi_h, gj_h, wi_h, bi_h, wj_h, bj_h, out_h,
          xi_v, xj_v, wi_v, bi_v, wj_v, bj_v, zi_v, zj_v, gi_b, gj_b, ob,
          sem_i, sem_j, sem_gi, sem_gj, sem_o):
    nkb = gi_h.shape[0] // BK

    def gi_cp(idx):
        return pltpu.make_async_copy(
            gi_h.at[pl.ds(idx * BK, BK)], gi_b.at[idx % NBUF],
            sem_gi.at[idx % NBUF])

    def gj_cp(idx):
        return pltpu.make_async_copy(
            gj_h.at[pl.ds(idx * BK, BK)], gj_b.at[idx % NBUF],
            sem_gj.at[idx % NBUF])

    def out_cp(idx):
        return pltpu.make_async_copy(
            ob.at[idx % 2], out_h.at[pl.ds(idx * BK, BK)], sem_o.at[idx % 2])

    cp_wi = pltpu.make_async_copy(wi_h, wi_v, sem_i)
    cp_bi = pltpu.make_async_copy(bi_h, bi_v, sem_i)
    cp_xi = pltpu.make_async_copy(xi_h, xi_v, sem_i)
    cp_wj = pltpu.make_async_copy(wj_h, wj_v, sem_j)
    cp_bj = pltpu.make_async_copy(bj_h, bj_v, sem_j)
    cp_xj = pltpu.make_async_copy(xj_h, xj_v, sem_j)
    cp_wi.start(); cp_bi.start(); cp_xi.start()
    cp_wj.start(); cp_bj.start(); cp_xj.start()
    gj_cp(0).start(); gi_cp(0).start()
    gj_cp(1).start(); gi_cp(1).start()

    cp_wi.wait(); cp_bi.wait(); cp_xi.wait()
    zi_v[...] = (
        jnp.dot(xi_v[...], wi_v[...], preferred_element_type=jnp.float32)
        + bi_v[...]
    )
    cp_wj.wait(); cp_bj.wait(); cp_xj.wait()
    zj_v[...] = (
        jnp.dot(xj_v[...], wj_v[...], preferred_element_type=jnp.float32)
        + bj_v[...]
    )

    for i in range(nkb):
        if i + 2 < nkb:
            gj_cp(i + 2).start()
            gi_cp(i + 2).start()
        slot = i % NBUF
        oslot = i % 2
        gj_cp(i).wait()
        acc = jnp.dot(gj_b[slot], zj_v[...], preferred_element_type=jnp.float32)
        gi_cp(i).wait()
        acc += jnp.dot(gi_b[slot], zi_v[...], preferred_element_type=jnp.float32)
        if i >= 2:
            out_cp(i - 2).wait()
        ob[oslot] = jnp.maximum(acc, 0.0)
        out_cp(i).start()

    out_cp(nkb - 2).wait()
    out_cp(nkb - 1).wait()


@jax.jit
def kernel(xi, xj, Gi2k, Gj2k, Wi, bi, Wj, bj):
    n_k = Gi2k.shape[0]
    n_i, ci = xi.shape
    n_j, cj = xj.shape
    ck = Wi.shape[1]

    any_spec = pl.BlockSpec(memory_space=pl.ANY)
    out = pl.pallas_call(
        _body,
        in_specs=[any_spec] * 8,
        out_specs=any_spec,
        out_shape=jax.ShapeDtypeStruct((n_k, ck), jnp.float32),
        scratch_shapes=[
            pltpu.VMEM((n_i, ci), jnp.float32),
            pltpu.VMEM((n_j, cj), jnp.float32),
            pltpu.VMEM((ci, ck), jnp.float32),
            pltpu.VMEM((1, ck), jnp.float32),
            pltpu.VMEM((cj, ck), jnp.float32),
            pltpu.VMEM((1, ck), jnp.float32),
            pltpu.VMEM((n_i, ck), jnp.float32),
            pltpu.VMEM((n_j, ck), jnp.float32),
            pltpu.VMEM((NBUF, BK, n_i), jnp.float32),
            pltpu.VMEM((NBUF, BK, n_j), jnp.float32),
            pltpu.VMEM((2, BK, ck), jnp.float32),
            pltpu.SemaphoreType.DMA,
            pltpu.SemaphoreType.DMA,
            pltpu.SemaphoreType.DMA((NBUF,)),
            pltpu.SemaphoreType.DMA((NBUF,)),
            pltpu.SemaphoreType.DMA((2,)),
        ],
        compiler_params=pltpu.CompilerParams(
            vmem_limit_bytes=110 * 1024 * 1024,
        ),
    )(xi, xj, Gi2k, Gj2k, Wi, bi.reshape(1, ck), Wj, bj.reshape(1, ck))
    return out
